# baseline (speedup 1.0000x reference)
import functools

import numpy as np

import jax
import jax.numpy as jnp
from jax import lax
from jax.experimental import pallas as pl
from jax.experimental.pallas import tpu as pltpu
from jax.experimental.pallas import tpu_sc as plsc

N = 10000
E = 320000
D = 128

NC = 2
NS = 16
NW = NC * NS
EPT = E // NW
K = 80
NCHUNKS = EPT // K
NTRIPLES = NCHUNKS // 3
RPT = 624
ZR = 16


def _sc_aggregate(x, src, dst, adj):
    mesh = plsc.VectorSubcoreMesh(core_axis_name="c", subcore_axis_name="s")

    @functools.partial(
        pl.kernel,
        out_type=jax.ShapeDtypeStruct((NC, N, D), jnp.float32),
        mesh=mesh,
        compiler_params=pltpu.CompilerParams(needs_layout_passes=False,
                                             use_tc_tiling_on_sc=False),
        scratch_types=[
            pltpu.VMEM_SHARED((N, D), jnp.float32),
            pltpu.VMEM((K,), jnp.int32),
            pltpu.VMEM((K,), jnp.int32),
            pltpu.VMEM((K,), jnp.int32),
            pltpu.VMEM((K,), jnp.int32),
            pltpu.VMEM((K,), jnp.int32),
            pltpu.VMEM((K,), jnp.int32),
            pltpu.VMEM((K,), jnp.float32),
            pltpu.VMEM((K,), jnp.float32),
            pltpu.VMEM((K,), jnp.float32),
            pltpu.VMEM((K, D), jnp.float32),
            pltpu.VMEM((K, D), jnp.float32),
            pltpu.VMEM((K, D), jnp.float32),
            pltpu.VMEM((K, D // 2), jnp.int32),
            pltpu.VMEM((K, D // 2), jnp.int32),
            pltpu.VMEM((K, D // 2), jnp.int32),
            pltpu.VMEM((ZR, D), jnp.float32),
            pltpu.SemaphoreType.DMA,
            pltpu.SemaphoreType.DMA,
            pltpu.SemaphoreType.DMA,
            pltpu.SemaphoreType.DMA,
            pltpu.SemaphoreType.DMA,
            pltpu.SemaphoreType.DMA,
            pltpu.SemaphoreType.DMA,
            pltpu.SemaphoreType.DMA,
            pltpu.SemaphoreType.DMA,
            pltpu.SemaphoreType.DMA,
            pltpu.SemaphoreType.DMA,
            pltpu.SemaphoreType.DMA,
        ],
    )
    def agg(x_hbm, src_hbm, dst_hbm, adj_hbm, out_hbm,
            acc, isrc0, isrc1, isrc2, idst0, idst1, idst2, av0, av1, av2,
            rows0, rows1, rows2, rbf0, rbf1, rbf2, zbuf,
            isem0, isem1, isem2, dsem0, dsem1, dsem2,
            gsem0, gsem1, gsem2, ssem0, ssem1, ssem2):
        cid = lax.axis_index("c")
        sid = lax.axis_index("s")
        wid = cid * NS + sid
        base0 = wid * EPT

        isrc = (isrc0, isrc1, isrc2)
        idst = (idst0, idst1, idst2)
        av = (av0, av1, av2)
        rows = (rows0, rows1, rows2)
        rbf = (rbf0, rbf1, rbf2)
        isem = (isem0, isem1, isem2)
        dsem = (dsem0, dsem1, dsem2)
        gsem = (gsem0, gsem1, gsem2)
        ssem = (ssem0, ssem1, ssem2)

        zv = jnp.zeros((16,), jnp.float32)

        def zrow(i, _):
            for j in range(D // 16):
                zbuf[i, pl.ds(j * 16, 16)] = zv
            return ()

        lax.fori_loop(0, ZR, zrow, ())

        def zcopy(i, _):
            pltpu.sync_copy(zbuf, acc.at[pl.ds(sid * RPT + i * ZR, ZR)])
            return ()

        lax.fori_loop(0, RPT // ZR, zcopy, ())

        @pl.when(sid == NS - 1)
        def _zero_tail():
            pltpu.sync_copy(zbuf, acc.at[pl.ds(NS * RPT, ZR)])

        plsc.subcore_barrier()

        def _base(c):
            return jnp.minimum(base0 + c * K, E - K)

        def issue_idx(c, b):
            base = _base(c)
            pltpu.async_copy(src_hbm.at[pl.ds(base, K)], isrc[b], isem[b])
            pltpu.async_copy(adj_hbm.at[pl.ds(base, K)], av[b], isem[b])

        def wait_idx(b):
            pltpu.make_async_copy(src_hbm.at[pl.ds(0, K)], isrc[b], isem[b]).wait()
            pltpu.make_async_copy(adj_hbm.at[pl.ds(0, K)], av[b], isem[b]).wait()

        def issue_idst(c, b):
            pltpu.async_copy(dst_hbm.at[pl.ds(_base(c), K)], idst[b], dsem[b])

        def wait_idst(b):
            pltpu.make_async_copy(dst_hbm.at[pl.ds(0, K)], idst[b], dsem[b]).wait()

        def issue_gather(b):
            pltpu.async_copy(x_hbm.at[isrc[b]], rbf[b], gsem[b])

        def wait_gather(b):
            pltpu.make_async_copy(x_hbm.at[isrc[b]], rbf[b], gsem[b]).wait()

        def issue_scatter(b):
            pltpu.async_copy(rows[b], acc.at[idst[b]], ssem[b], add=True)

        def wait_scatter(b):
            pltpu.make_async_copy(rows[b], acc.at[idst[b]], ssem[b]).wait()

        def compute(b):
            rb, ab, fb = rbf[b], av[b], rows[b]
            himask = jnp.full((16,), -65536, jnp.int32)

            def group(t, _):
                a16 = ab[pl.ds(t * 16, 16)]
                for i in range(16):
                    a = jnp.broadcast_to(a16[i], (16,))
                    k = t * 16 + i
                    for blk in range(D // 32):
                        w = rb[k, pl.ds(blk * 16, 16)]
                        lo = plsc.bitcast(w << 16, jnp.float32)
                        hi = plsc.bitcast(w & himask, jnp.float32)
                        fb[k, pl.ds(blk * 32, 16)] = lo * a
                        fb[k, pl.ds(blk * 32 + 16, 16)] = hi * a
                return ()

            lax.fori_loop(0, K // 16, group, ())

        def step(c, b, first):
            bn = (b + 1) % 3
            wait_gather(b)
            wait_idx(bn)
            if not first:
                wait_scatter(bn)

            @pl.when(c + 1 < NCHUNKS)
            def _g():
                issue_gather(bn)

            issue_idst(c + 1, bn)
            compute(b)
            issue_idx(c + 3, b)
            wait_idst(b)
            issue_scatter(b)

        issue_idx(0, 0)
        issue_idx(1, 1)
        issue_idx(2, 2)
        issue_idst(0, 0)
        wait_idx(0)
        issue_gather(0)

        step(0, 0, True)
        step(1, 1, True)
        step(2, 2, False)

        def triple(p, _):
            c0 = 3 * p
            step(c0, 0, False)
            step(c0 + 1, 1, False)
            step(c0 + 2, 2, False)
            return ()

        lax.fori_loop(1, NTRIPLES, triple, ())

        step(NCHUNKS - 2, 0, False)
        step(NCHUNKS - 1, 1, False)
        wait_scatter(0)
        wait_scatter(1)
        wait_idx(0)
        wait_idx(1)
        wait_idst(2)
        plsc.subcore_barrier()

        pltpu.sync_copy(acc.at[pl.ds(sid * RPT, RPT)],
                        out_hbm.at[cid, pl.ds(sid * RPT, RPT)])

        @pl.when(sid == NS - 1)
        def _copy_tail():
            pltpu.sync_copy(acc.at[pl.ds(NS * RPT, N - NS * RPT)],
                            out_hbm.at[cid, pl.ds(NS * RPT, N - NS * RPT)])

    return agg(x, src, dst, adj)


def _tc_body(p_ref, w_ref, o_ref):
    s = p_ref[0] + p_ref[1]
    h = lax.dot_general(s, w_ref[...], (((1,), (1,)), ((), ())),
                        preferred_element_type=jnp.float32,
                        precision=lax.Precision.HIGHEST)
    o_ref[...] = jnp.maximum(h, 0.0)


def _tc_combine_matmul_relu(partials, W):
    bm = 1000
    return pl.pallas_call(
        _tc_body,
        grid=(N // bm,),
        in_specs=[
            pl.BlockSpec((NC, bm, D), lambda i: (0, i, 0)),
            pl.BlockSpec((D, D), lambda i: (0, 0)),
        ],
        out_specs=pl.BlockSpec((bm, D), lambda i: (i, 0)),
        out_shape=jax.ShapeDtypeStruct((N, D), jnp.float32),
    )(partials, W)


_G = np.empty(D, np.int32)
for _t in range(D // 32):
    for _l in range(16):
        _G[32 * _t + _l] = 32 * _t + 2 * _l
        _G[32 * _t + 16 + _l] = 32 * _t + 2 * _l + 1


def kernel(x, edge_index, adj_values, W):
    dst = edge_index[0]
    src = edge_index[1]
    x_pairs = lax.bitcast_convert_type(
        x.astype(jnp.bfloat16).reshape(N, D // 2, 2), jnp.int32)
    partials = _sc_aggregate(x_pairs, src, dst, adj_values)
    return _tc_combine_matmul_relu(partials, W[:, _G])


# speedup vs baseline: 1.7239x; 1.7239x over previous
import functools

import jax
import jax.numpy as jnp
from jax import lax
from jax.experimental import pallas as pl
from jax.experimental.pallas import tpu as pltpu
from jax.experimental.pallas import tpu_sc as plsc

N = 10000
E = 320000
D = 128

NC = 2
NS = 16
NW = NC * NS
EPT = E // NW
K = 80
KH = K // 2
NCHUNKS = EPT // K
NTRIPLES = NCHUNKS // 3
RPT = 624
ZR = 16


def _sc_aggregate(x, src, dst, adj):
    mesh = plsc.VectorSubcoreMesh(core_axis_name="c", subcore_axis_name="s")

    @functools.partial(
        pl.kernel,
        out_type=jax.ShapeDtypeStruct((NC, N, D), jnp.float32),
        mesh=mesh,
        scratch_types=[
            pltpu.VMEM_SHARED((N, D), jnp.float32),
            pltpu.VMEM((K,), jnp.int32),
            pltpu.VMEM((K,), jnp.int32),
            pltpu.VMEM((K,), jnp.int32),
            pltpu.VMEM((K,), jnp.int32),
            pltpu.VMEM((K,), jnp.int32),
            pltpu.VMEM((K,), jnp.int32),
            pltpu.VMEM((K,), jnp.float32),
            pltpu.VMEM((K,), jnp.float32),
            pltpu.VMEM((K,), jnp.float32),
            pltpu.VMEM((K, D), jnp.float32),
            pltpu.VMEM((K, D), jnp.float32),
            pltpu.VMEM((K, D), jnp.float32),
            pltpu.VMEM((ZR, D), jnp.float32),
            pltpu.SemaphoreType.DMA,
            pltpu.SemaphoreType.DMA,
            pltpu.SemaphoreType.DMA,
            pltpu.SemaphoreType.DMA,
            pltpu.SemaphoreType.DMA,
            pltpu.SemaphoreType.DMA,
            pltpu.SemaphoreType.DMA,
            pltpu.SemaphoreType.DMA,
            pltpu.SemaphoreType.DMA,
            pltpu.SemaphoreType.DMA,
            pltpu.SemaphoreType.DMA,
            pltpu.SemaphoreType.DMA,
            pltpu.SemaphoreType.DMA,
            pltpu.SemaphoreType.DMA,
            pltpu.SemaphoreType.DMA,
        ],
    )
    def agg(x_hbm, src_hbm, dst_hbm, adj_hbm, out_hbm,
            acc, isrc0, isrc1, isrc2, idst0, idst1, idst2, av0, av1, av2,
            rows0, rows1, rows2, zbuf,
            isem0, isem1, isem2, dsem0, dsem1, dsem2,
            gsa0, gsa1, gsa2, gsb0, gsb1, gsb2, ssem0, ssem1, ssem2):
        cid = lax.axis_index("c")
        sid = lax.axis_index("s")
        wid = cid * NS + sid
        base0 = wid * EPT

        isrc = (isrc0, isrc1, isrc2)
        idst = (idst0, idst1, idst2)
        av = (av0, av1, av2)
        rows = (rows0, rows1, rows2)
        isem = (isem0, isem1, isem2)
        dsem = (dsem0, dsem1, dsem2)
        gsa = (gsa0, gsa1, gsa2)
        gsb = (gsb0, gsb1, gsb2)
        ssem = (ssem0, ssem1, ssem2)

        zv = jnp.zeros((16,), jnp.float32)

        def zrow(i, _):
            for j in range(D // 16):
                zbuf[i, pl.ds(j * 16, 16)] = zv
            return ()

        lax.fori_loop(0, ZR, zrow, ())

        def zcopy(i, _):
            pltpu.sync_copy(zbuf, acc.at[pl.ds(sid * RPT + i * ZR, ZR)])
            return ()

        lax.fori_loop(0, RPT // ZR, zcopy, ())

        @pl.when(sid == NS - 1)
        def _zero_tail():
            pltpu.sync_copy(zbuf, acc.at[pl.ds(NS * RPT, ZR)])

        plsc.subcore_barrier()

        def _base(c):
            return jnp.minimum(base0 + c * K, E - K)

        def issue_idx(c, b):
            base = _base(c)
            pltpu.async_copy(src_hbm.at[pl.ds(base, K)], isrc[b], isem[b])
            pltpu.async_copy(adj_hbm.at[pl.ds(base, K)], av[b], isem[b])

        def wait_idx(b):
            pltpu.make_async_copy(src_hbm.at[pl.ds(0, K)], isrc[b], isem[b]).wait()
            pltpu.make_async_copy(adj_hbm.at[pl.ds(0, K)], av[b], isem[b]).wait()

        def issue_idst(c, b):
            pltpu.async_copy(dst_hbm.at[pl.ds(_base(c), K)], idst[b], dsem[b])

        def wait_idst(b):
            pltpu.make_async_copy(dst_hbm.at[pl.ds(0, K)], idst[b], dsem[b]).wait()

        def issue_gather(b):
            pltpu.async_copy(x_hbm.at[isrc[b].at[pl.ds(0, KH)]],
                             rows[b].at[pl.ds(0, KH)], gsa[b])
            pltpu.async_copy(x_hbm.at[isrc[b].at[pl.ds(KH, KH)]],
                             rows[b].at[pl.ds(KH, KH)], gsb[b])

        def wait_gather(b):
            pltpu.make_async_copy(x_hbm.at[isrc[b].at[pl.ds(0, KH)]],
                                  rows[b].at[pl.ds(0, KH)], gsa[b]).wait()
            pltpu.make_async_copy(x_hbm.at[isrc[b].at[pl.ds(KH, KH)]],
                                  rows[b].at[pl.ds(KH, KH)], gsb[b]).wait()

        def issue_scatter(b):
            pltpu.async_copy(rows[b], acc.at[idst[b]], ssem[b], add=True)

        def wait_scatter(b):
            pltpu.make_async_copy(rows[b], acc.at[idst[b]], ssem[b]).wait()

        def compute(b):
            rb, ab = rows[b], av[b]

            def group(t, _):
                a16 = ab[pl.ds(t * 16, 16)]
                for i in range(16):
                    a = jnp.broadcast_to(a16[i], (16,))
                    k = t * 16 + i
                    for j in range(D // 16):
                        sl = pl.ds(j * 16, 16)
                        rb[k, sl] = rb[k, sl] * a
                return ()

            lax.fori_loop(0, K // 16, group, ())

        def step(c, b, first):
            bn = (b + 1) % 3
            wait_gather(b)
            wait_idx(bn)
            if not first:
                wait_scatter(bn)

            @pl.when(c + 1 < NCHUNKS)
            def _g():
                issue_gather(bn)

            issue_idst(c + 1, bn)
            compute(b)
            issue_idx(c + 3, b)
            wait_idst(b)
            issue_scatter(b)

        issue_idx(0, 0)
        issue_idx(1, 1)
        issue_idx(2, 2)
        issue_idst(0, 0)
        wait_idx(0)
        issue_gather(0)

        step(0, 0, True)
        step(1, 1, True)
        step(2, 2, False)

        def triple(p, _):
            c0 = 3 * p
            step(c0, 0, False)
            step(c0 + 1, 1, False)
            step(c0 + 2, 2, False)
            return ()

        lax.fori_loop(1, NTRIPLES, triple, ())

        step(NCHUNKS - 2, 0, False)
        step(NCHUNKS - 1, 1, False)
        wait_scatter(0)
        wait_scatter(1)
        wait_idx(0)
        wait_idx(1)
        wait_idst(2)
        plsc.subcore_barrier()

        pltpu.sync_copy(acc.at[pl.ds(sid * RPT, RPT)],
                        out_hbm.at[cid, pl.ds(sid * RPT, RPT)])

        @pl.when(sid == NS - 1)
        def _copy_tail():
            pltpu.sync_copy(acc.at[pl.ds(NS * RPT, N - NS * RPT)],
                            out_hbm.at[cid, pl.ds(NS * RPT, N - NS * RPT)])

    return agg(x, src, dst, adj)


def _tc_body(p_ref, w_ref, o_ref):
    s = p_ref[0] + p_ref[1]
    h = lax.dot_general(s, w_ref[...], (((1,), (1,)), ((), ())),
                        preferred_element_type=jnp.float32,
                        precision=lax.Precision.HIGHEST)
    o_ref[...] = jnp.maximum(h, 0.0)


def _tc_combine_matmul_relu(partials, W):
    bm = 1000
    return pl.pallas_call(
        _tc_body,
        grid=(N // bm,),
        in_specs=[
            pl.BlockSpec((NC, bm, D), lambda i: (0, i, 0)),
            pl.BlockSpec((D, D), lambda i: (0, 0)),
        ],
        out_specs=pl.BlockSpec((bm, D), lambda i: (i, 0)),
        out_shape=jax.ShapeDtypeStruct((N, D), jnp.float32),
    )(partials, W)


def kernel(x, edge_index, adj_values, W):
    dst = edge_index[0]
    src = edge_index[1]
    partials = _sc_aggregate(x, src, dst, adj_values)
    return _tc_combine_matmul_relu(partials, W)


# speedup vs baseline: 1.7253x; 1.0008x over previous
import functools

import jax
import jax.numpy as jnp
from jax import lax
from jax.experimental import pallas as pl
from jax.experimental.pallas import tpu as pltpu
from jax.experimental.pallas import tpu_sc as plsc

N = 10000
E = 320000
D = 128

NC = 2
NS = 16
NW = NC * NS
EPT = E // NW
K = 80
KQ = K // 4
NCHUNKS = EPT // K
NTRIPLES = NCHUNKS // 3
RPT = 624
ZR = 16


def _sc_aggregate(x, src, dst, adj):
    mesh = plsc.VectorSubcoreMesh(core_axis_name="c", subcore_axis_name="s")

    @functools.partial(
        pl.kernel,
        out_type=jax.ShapeDtypeStruct((NC, N, D), jnp.float32),
        mesh=mesh,
        scratch_types=[
            pltpu.VMEM_SHARED((N, D), jnp.float32),
            pltpu.VMEM((K,), jnp.int32),
            pltpu.VMEM((K,), jnp.int32),
            pltpu.VMEM((K,), jnp.int32),
            pltpu.VMEM((K,), jnp.int32),
            pltpu.VMEM((K,), jnp.int32),
            pltpu.VMEM((K,), jnp.int32),
            pltpu.VMEM((K,), jnp.float32),
            pltpu.VMEM((K,), jnp.float32),
            pltpu.VMEM((K,), jnp.float32),
            pltpu.VMEM((K, D), jnp.float32),
            pltpu.VMEM((K, D), jnp.float32),
            pltpu.VMEM((K, D), jnp.float32),
            pltpu.VMEM((ZR, D), jnp.float32),
            pltpu.SemaphoreType.DMA,
            pltpu.SemaphoreType.DMA,
            pltpu.SemaphoreType.DMA,
            pltpu.SemaphoreType.DMA,
            pltpu.SemaphoreType.DMA,
            pltpu.SemaphoreType.DMA,
            pltpu.SemaphoreType.DMA,
            pltpu.SemaphoreType.DMA,
            pltpu.SemaphoreType.DMA,
            pltpu.SemaphoreType.DMA,
            pltpu.SemaphoreType.DMA,
            pltpu.SemaphoreType.DMA,
            pltpu.SemaphoreType.DMA,
            pltpu.SemaphoreType.DMA,
            pltpu.SemaphoreType.DMA,
        ],
    )
    def agg(x_hbm, src_hbm, dst_hbm, adj_hbm, out_hbm,
            acc, isrc0, isrc1, isrc2, idst0, idst1, idst2, av0, av1, av2,
            rows0, rows1, rows2, zbuf,
            isem0, isem1, isem2, dsem0, dsem1, dsem2,
            gsa0, gsa1, gsa2, gsb0, gsb1, gsb2, ssem0, ssem1, ssem2):
        cid = lax.axis_index("c")
        sid = lax.axis_index("s")
        wid = cid * NS + sid
        base0 = wid * EPT

        isrc = (isrc0, isrc1, isrc2)
        idst = (idst0, idst1, idst2)
        av = (av0, av1, av2)
        rows = (rows0, rows1, rows2)
        isem = (isem0, isem1, isem2)
        dsem = (dsem0, dsem1, dsem2)
        gsa = (gsa0, gsa1, gsa2)
        gsb = (gsb0, gsb1, gsb2)
        ssem = (ssem0, ssem1, ssem2)

        def GSPLIT(b):
            return ((0, 24, gsa[b]), (24, 24, gsb[b]),
                    (48, 16, gsa[b]), (64, 16, gsb[b]))

        zv = jnp.zeros((16,), jnp.float32)

        def zrow(i, _):
            for j in range(D // 16):
                zbuf[i, pl.ds(j * 16, 16)] = zv
            return ()

        lax.fori_loop(0, ZR, zrow, ())

        def zcopy(i, _):
            pltpu.sync_copy(zbuf, acc.at[pl.ds(sid * RPT + i * ZR, ZR)])
            return ()

        lax.fori_loop(0, RPT // ZR, zcopy, ())

        @pl.when(sid == NS - 1)
        def _zero_tail():
            pltpu.sync_copy(zbuf, acc.at[pl.ds(NS * RPT, ZR)])

        plsc.subcore_barrier()

        def _base(c):
            return jnp.minimum(base0 + c * K, E - K)

        def issue_idx(c, b):
            base = _base(c)
            pltpu.async_copy(src_hbm.at[pl.ds(base, K)], isrc[b], isem[b])
            pltpu.async_copy(adj_hbm.at[pl.ds(base, K)], av[b], isem[b])

        def wait_idx(b):
            pltpu.make_async_copy(src_hbm.at[pl.ds(0, K)], isrc[b], isem[b]).wait()
            pltpu.make_async_copy(adj_hbm.at[pl.ds(0, K)], av[b], isem[b]).wait()

        def issue_idst(c, b):
            pltpu.async_copy(dst_hbm.at[pl.ds(_base(c), K)], idst[b], dsem[b])

        def wait_idst(b):
            pltpu.make_async_copy(dst_hbm.at[pl.ds(0, K)], idst[b], dsem[b]).wait()

        def issue_gather(b):
            for o, n, sem in GSPLIT(b):
                pltpu.async_copy(x_hbm.at[isrc[b].at[pl.ds(o, n)]],
                                 rows[b].at[pl.ds(o, n)], sem)

        def wait_gather(b):
            for o, n, sem in GSPLIT(b):
                pltpu.make_async_copy(x_hbm.at[isrc[b].at[pl.ds(o, n)]],
                                      rows[b].at[pl.ds(o, n)], sem).wait()

        def issue_scatter(b):
            pltpu.async_copy(rows[b], acc.at[idst[b]], ssem[b], add=True)

        def wait_scatter(b):
            pltpu.make_async_copy(rows[b], acc.at[idst[b]], ssem[b]).wait()

        def compute(b):
            rb, ab = rows[b], av[b]

            def group(t, _):
                a16 = ab[pl.ds(t * 16, 16)]
                for i in range(16):
                    a = jnp.broadcast_to(a16[i], (16,))
                    k = t * 16 + i
                    for j in range(D // 16):
                        sl = pl.ds(j * 16, 16)
                        rb[k, sl] = rb[k, sl] * a
                return ()

            lax.fori_loop(0, K // 16, group, ())

        def step(c, b, first):
            bn = (b + 1) % 3
            wait_gather(b)
            wait_idx(bn)
            if not first:
                wait_scatter(bn)

            @pl.when(c + 1 < NCHUNKS)
            def _g():
                issue_gather(bn)

            issue_idst(c + 1, bn)
            compute(b)
            issue_idx(c + 3, b)
            wait_idst(b)
            issue_scatter(b)

        issue_idx(0, 0)
        issue_idx(1, 1)
        issue_idx(2, 2)
        issue_idst(0, 0)
        wait_idx(0)
        issue_gather(0)

        step(0, 0, True)
        step(1, 1, True)
        step(2, 2, False)

        def triple(p, _):
            c0 = 3 * p
            step(c0, 0, False)
            step(c0 + 1, 1, False)
            step(c0 + 2, 2, False)
            return ()

        lax.fori_loop(1, NTRIPLES, triple, ())

        step(NCHUNKS - 2, 0, False)
        step(NCHUNKS - 1, 1, False)
        wait_scatter(0)
        wait_scatter(1)
        wait_idx(0)
        wait_idx(1)
        wait_idst(2)
        plsc.subcore_barrier()

        pltpu.sync_copy(acc.at[pl.ds(sid * RPT, RPT)],
                        out_hbm.at[cid, pl.ds(sid * RPT, RPT)])

        @pl.when(sid == NS - 1)
        def _copy_tail():
            pltpu.sync_copy(acc.at[pl.ds(NS * RPT, N - NS * RPT)],
                            out_hbm.at[cid, pl.ds(NS * RPT, N - NS * RPT)])

    return agg(x, src, dst, adj)


def _tc_body(p_ref, w_ref, o_ref):
    s = p_ref[0] + p_ref[1]
    h = lax.dot_general(s, w_ref[...], (((1,), (1,)), ((), ())),
                        preferred_element_type=jnp.float32,
                        precision=lax.Precision.HIGHEST)
    o_ref[...] = jnp.maximum(h, 0.0)


def _tc_combine_matmul_relu(partials, W):
    bm = 1000
    return pl.pallas_call(
        _tc_body,
        grid=(N // bm,),
        in_specs=[
            pl.BlockSpec((NC, bm, D), lambda i: (0, i, 0)),
            pl.BlockSpec((D, D), lambda i: (0, 0)),
        ],
        out_specs=pl.BlockSpec((bm, D), lambda i: (i, 0)),
        out_shape=jax.ShapeDtypeStruct((N, D), jnp.float32),
    )(partials, W)


def kernel(x, edge_index, adj_values, W):
    dst = edge_index[0]
    src = edge_index[1]
    partials = _sc_aggregate(x, src, dst, adj_values)
    return _tc_combine_matmul_relu(partials, W)


# speedup vs baseline: 1.7662x; 1.0237x over previous
import functools

import jax
import jax.numpy as jnp
from jax import lax
from jax.experimental import pallas as pl
from jax.experimental.pallas import tpu as pltpu
from jax.experimental.pallas import tpu_sc as plsc

N = 10000
E = 320000
D = 128

NC = 2
NS = 16
NW = NC * NS
EPT = E // NW
K = 80
KQ = K // 4
NCHUNKS = EPT // K
NTRIPLES = NCHUNKS // 3
RPT = 624
ZR = 16


def _sc_aggregate(x, src, dst, adj):
    mesh = plsc.VectorSubcoreMesh(core_axis_name="c", subcore_axis_name="s")

    @functools.partial(
        pl.kernel,
        out_type=jax.ShapeDtypeStruct((NC, N, D), jnp.float32),
        mesh=mesh,
        scratch_types=[
            pltpu.VMEM_SHARED((N, D), jnp.float32),
            pltpu.VMEM((K,), jnp.int32),
            pltpu.VMEM((K,), jnp.int32),
            pltpu.VMEM((K,), jnp.int32),
            pltpu.VMEM((K,), jnp.int32),
            pltpu.VMEM((K,), jnp.int32),
            pltpu.VMEM((K,), jnp.int32),
            pltpu.VMEM((K,), jnp.float32),
            pltpu.VMEM((K,), jnp.float32),
            pltpu.VMEM((K,), jnp.float32),
            pltpu.VMEM((K, D), jnp.float32),
            pltpu.VMEM((K, D), jnp.float32),
            pltpu.VMEM((K, D), jnp.float32),
            pltpu.VMEM((ZR, D), jnp.float32),
            pltpu.SemaphoreType.DMA,
            pltpu.SemaphoreType.DMA,
            pltpu.SemaphoreType.DMA,
            pltpu.SemaphoreType.DMA,
            pltpu.SemaphoreType.DMA,
            pltpu.SemaphoreType.DMA,
            pltpu.SemaphoreType.DMA,
            pltpu.SemaphoreType.DMA,
            pltpu.SemaphoreType.DMA,
            pltpu.SemaphoreType.DMA,
            pltpu.SemaphoreType.DMA,
            pltpu.SemaphoreType.DMA,
            pltpu.SemaphoreType.DMA,
            pltpu.SemaphoreType.DMA,
            pltpu.SemaphoreType.DMA,
        ],
    )
    def agg(x_hbm, src_hbm, dst_hbm, adj_hbm, out_hbm,
            acc, isrc0, isrc1, isrc2, idst0, idst1, idst2, av0, av1, av2,
            rows0, rows1, rows2, zbuf,
            isem0, isem1, isem2, dsem0, dsem1, dsem2,
            gsa0, gsa1, gsa2, gsb0, gsb1, gsb2, ssem0, ssem1, ssem2):
        cid = lax.axis_index("c")
        sid = lax.axis_index("s")
        wid = cid * NS + sid
        base0 = wid * EPT

        isrc = (isrc0, isrc1, isrc2)
        idst = (idst0, idst1, idst2)
        av = (av0, av1, av2)
        rows = (rows0, rows1, rows2)
        isem = (isem0, isem1, isem2)
        dsem = (dsem0, dsem1, dsem2)
        gsa = (gsa0, gsa1, gsa2)
        gsb = (gsb0, gsb1, gsb2)
        ssem = (ssem0, ssem1, ssem2)

        def GSPLIT(b):
            return ((0, 24, gsa[b]), (24, 24, gsb[b]),
                    (48, 16, gsa[b]), (64, 16, gsb[b]))

        zv = jnp.zeros((16,), jnp.float32)

        def zrow(i, _):
            for j in range(D // 16):
                zbuf[i, pl.ds(j * 16, 16)] = zv
            return ()

        lax.fori_loop(0, ZR, zrow, ())

        def zcopy(i, _):
            pltpu.sync_copy(zbuf, acc.at[pl.ds(sid * RPT + i * ZR, ZR)])
            return ()

        lax.fori_loop(0, RPT // ZR, zcopy, ())

        @pl.when(sid == NS - 1)
        def _zero_tail():
            pltpu.sync_copy(zbuf, acc.at[pl.ds(NS * RPT, ZR)])

        plsc.subcore_barrier()

        def _base(c):
            return jnp.minimum(base0 + c * K, E - K)

        def issue_idx(c, b):
            base = _base(c)
            pltpu.async_copy(src_hbm.at[pl.ds(base, K)], isrc[b], isem[b])
            pltpu.async_copy(adj_hbm.at[pl.ds(base, K)], av[b], isem[b])

        def wait_idx(b):
            pltpu.make_async_copy(src_hbm.at[pl.ds(0, K)], isrc[b], isem[b]).wait()
            pltpu.make_async_copy(adj_hbm.at[pl.ds(0, K)], av[b], isem[b]).wait()

        def issue_idst(c, b):
            pltpu.async_copy(dst_hbm.at[pl.ds(_base(c), K)], idst[b], dsem[b])

        def wait_idst(b):
            pltpu.make_async_copy(dst_hbm.at[pl.ds(0, K)], idst[b], dsem[b]).wait()

        def issue_gather(b):
            for o, n, sem in GSPLIT(b):
                pltpu.async_copy(x_hbm.at[isrc[b].at[pl.ds(o, n)]],
                                 rows[b].at[pl.ds(o, n)], sem)

        def wait_gather(b):
            for o, n, sem in GSPLIT(b):
                pltpu.make_async_copy(x_hbm.at[isrc[b].at[pl.ds(o, n)]],
                                      rows[b].at[pl.ds(o, n)], sem).wait()

        def issue_scatter(b):
            pltpu.async_copy(rows[b], acc.at[idst[b]], ssem[b], add=True)

        def wait_scatter(b):
            pltpu.make_async_copy(rows[b], acc.at[idst[b]], ssem[b]).wait()

        def compute(b):
            rb, ab = rows[b], av[b]

            def group(t, _):
                a16 = ab[pl.ds(t * 16, 16)]
                for i in range(16):
                    a = jnp.broadcast_to(a16[i], (16,))
                    k = t * 16 + i
                    for j in range(D // 16):
                        sl = pl.ds(j * 16, 16)
                        rb[k, sl] = rb[k, sl] * a
                return ()

            lax.fori_loop(0, K // 16, group, ())

        def step(c, b, first):
            bn = (b + 1) % 3
            wait_gather(b)
            wait_idx(bn)
            if not first:
                wait_scatter(bn)

            @pl.when(c + 1 < NCHUNKS)
            def _g():
                issue_gather(bn)

            issue_idst(c + 1, bn)
            compute(b)
            issue_idx(c + 3, b)
            wait_idst(b)
            issue_scatter(b)

        issue_idx(0, 0)
        issue_idx(1, 1)
        issue_idx(2, 2)
        issue_idst(0, 0)
        wait_idx(0)
        issue_gather(0)

        step(0, 0, True)
        step(1, 1, True)
        step(2, 2, False)

        def triple(p, _):
            c0 = 3 * p
            step(c0, 0, False)
            step(c0 + 1, 1, False)
            step(c0 + 2, 2, False)
            return ()

        lax.fori_loop(1, NTRIPLES, triple, ())

        step(NCHUNKS - 2, 0, False)
        step(NCHUNKS - 1, 1, False)
        wait_scatter(0)
        wait_scatter(1)
        wait_idx(0)
        wait_idx(1)
        wait_idst(2)
        plsc.subcore_barrier()

        pltpu.sync_copy(acc.at[pl.ds(sid * RPT, RPT)],
                        out_hbm.at[cid, pl.ds(sid * RPT, RPT)])

        @pl.when(sid == NS - 1)
        def _copy_tail():
            pltpu.sync_copy(acc.at[pl.ds(NS * RPT, N - NS * RPT)],
                            out_hbm.at[cid, pl.ds(NS * RPT, N - NS * RPT)])

    return agg(x, src, dst, adj)


def _tc_body(p_ref, w_ref, o_ref):
    s = p_ref[0] + p_ref[1]
    h = lax.dot_general(s, w_ref[...], (((1,), (1,)), ((), ())),
                        preferred_element_type=jnp.float32,
                        precision=lax.Precision.HIGHEST)
    o_ref[...] = jnp.maximum(h, 0.0)


def _tc_combine_matmul_relu(partials, W):
    bm = 2000
    return pl.pallas_call(
        _tc_body,
        grid=(N // bm,),
        in_specs=[
            pl.BlockSpec((NC, bm, D), lambda i: (0, i, 0)),
            pl.BlockSpec((D, D), lambda i: (0, 0)),
        ],
        out_specs=pl.BlockSpec((bm, D), lambda i: (i, 0)),
        out_shape=jax.ShapeDtypeStruct((N, D), jnp.float32),
    )(partials, W)


def kernel(x, edge_index, adj_values, W):
    partials = _sc_aggregate(x, edge_index[1], edge_index[0], adj_values)
    return _tc_combine_matmul_relu(partials, W)


# speedup vs baseline: 1.7850x; 1.0106x over previous
import functools

import jax
import jax.numpy as jnp
from jax import lax
from jax.experimental import pallas as pl
from jax.experimental.pallas import tpu as pltpu
from jax.experimental.pallas import tpu_sc as plsc

N = 10000
E = 320000
D = 128

NC = 2
NS = 16
NW = NC * NS
EPT = E // NW
K = 80
KQ = K // 4
NCHUNKS = EPT // K
NTRIPLES = NCHUNKS // 3
RPT = 624
ZR = 16


def _sc_aggregate(x, src, dst, adj):
    mesh = plsc.VectorSubcoreMesh(core_axis_name="c", subcore_axis_name="s")

    @functools.partial(
        pl.kernel,
        out_type=jax.ShapeDtypeStruct((NC, N, D), jnp.float32),
        mesh=mesh,
        scratch_types=[
            pltpu.VMEM_SHARED((N, D), jnp.float32),
            pltpu.VMEM((K,), jnp.int32),
            pltpu.VMEM((K,), jnp.int32),
            pltpu.VMEM((K,), jnp.int32),
            pltpu.VMEM((K,), jnp.int32),
            pltpu.VMEM((K,), jnp.int32),
            pltpu.VMEM((K,), jnp.int32),
            pltpu.VMEM((K,), jnp.float32),
            pltpu.VMEM((K,), jnp.float32),
            pltpu.VMEM((K,), jnp.float32),
            pltpu.VMEM((K, D), jnp.float32),
            pltpu.VMEM((K, D), jnp.float32),
            pltpu.VMEM((K, D), jnp.float32),
            pltpu.VMEM((ZR, D), jnp.float32),
            pltpu.SemaphoreType.DMA,
            pltpu.SemaphoreType.DMA,
            pltpu.SemaphoreType.DMA,
            pltpu.SemaphoreType.DMA,
            pltpu.SemaphoreType.DMA,
            pltpu.SemaphoreType.DMA,
            pltpu.SemaphoreType.DMA,
            pltpu.SemaphoreType.DMA,
            pltpu.SemaphoreType.DMA,
            pltpu.SemaphoreType.DMA,
            pltpu.SemaphoreType.DMA,
            pltpu.SemaphoreType.DMA,
            pltpu.SemaphoreType.DMA,
            pltpu.SemaphoreType.DMA,
            pltpu.SemaphoreType.DMA,
        ],
    )
    def agg(x_hbm, src_hbm, dst_hbm, adj_hbm, out_hbm,
            acc, isrc0, isrc1, isrc2, idst0, idst1, idst2, av0, av1, av2,
            rows0, rows1, rows2, zbuf,
            isem0, isem1, isem2, dsem0, dsem1, dsem2,
            gsa0, gsa1, gsa2, gsb0, gsb1, gsb2, ssem0, ssem1, ssem2):
        cid = lax.axis_index("c")
        sid = lax.axis_index("s")
        wid = cid * NS + sid
        base0 = wid * EPT

        isrc = (isrc0, isrc1, isrc2)
        idst = (idst0, idst1, idst2)
        av = (av0, av1, av2)
        rows = (rows0, rows1, rows2)
        isem = (isem0, isem1, isem2)
        dsem = (dsem0, dsem1, dsem2)
        gsa = (gsa0, gsa1, gsa2)
        gsb = (gsb0, gsb1, gsb2)
        ssem = (ssem0, ssem1, ssem2)

        def GSPLIT(b):
            return ((0, 24, gsa[b]), (24, 24, gsb[b]),
                    (48, 16, gsa[b]), (64, 16, gsb[b]))

        zv = jnp.zeros((16,), jnp.float32)

        def zrow(i, _):
            for j in range(D // 16):
                zbuf[i, pl.ds(j * 16, 16)] = zv
            return ()

        lax.fori_loop(0, ZR, zrow, ())

        def zcopy(i, _):
            pltpu.async_copy(zbuf, acc.at[pl.ds(sid * RPT + i * ZR, ZR)],
                             gsa0)
            return ()

        lax.fori_loop(0, RPT // ZR, zcopy, ())

        @pl.when(sid == NS - 1)
        def _zero_tail():
            pltpu.async_copy(zbuf, acc.at[pl.ds(NS * RPT, ZR)], gsa0)

        def zdrain(i, _):
            pltpu.make_async_copy(zbuf, acc.at[pl.ds(0, ZR)], gsa0).wait()
            return ()

        lax.fori_loop(0, RPT // ZR, zdrain, ())

        @pl.when(sid == NS - 1)
        def _zero_tail_drain():
            pltpu.make_async_copy(zbuf, acc.at[pl.ds(0, ZR)], gsa0).wait()

        plsc.subcore_barrier()

        def _base(c):
            return jnp.minimum(base0 + c * K, E - K)

        def issue_idx(c, b):
            base = _base(c)
            pltpu.async_copy(src_hbm.at[pl.ds(base, K)], isrc[b], isem[b])
            pltpu.async_copy(adj_hbm.at[pl.ds(base, K)], av[b], isem[b])

        def wait_idx(b):
            pltpu.make_async_copy(src_hbm.at[pl.ds(0, K)], isrc[b], isem[b]).wait()
            pltpu.make_async_copy(adj_hbm.at[pl.ds(0, K)], av[b], isem[b]).wait()

        def issue_idst(c, b):
            pltpu.async_copy(dst_hbm.at[pl.ds(_base(c), K)], idst[b], dsem[b])

        def wait_idst(b):
            pltpu.make_async_copy(dst_hbm.at[pl.ds(0, K)], idst[b], dsem[b]).wait()

        def issue_gather(b):
            for o, n, sem in GSPLIT(b):
                pltpu.async_copy(x_hbm.at[isrc[b].at[pl.ds(o, n)]],
                                 rows[b].at[pl.ds(o, n)], sem)

        def wait_gather(b):
            for o, n, sem in GSPLIT(b):
                pltpu.make_async_copy(x_hbm.at[isrc[b].at[pl.ds(o, n)]],
                                      rows[b].at[pl.ds(o, n)], sem).wait()

        def issue_scatter(b):
            pltpu.async_copy(rows[b], acc.at[idst[b]], ssem[b], add=True)

        def wait_scatter(b):
            pltpu.make_async_copy(rows[b], acc.at[idst[b]], ssem[b]).wait()

        def compute(b):
            rb, ab = rows[b], av[b]

            def group(t, _):
                a16 = ab[pl.ds(t * 16, 16)]
                for i in range(16):
                    a = jnp.broadcast_to(a16[i], (16,))
                    k = t * 16 + i
                    for j in range(D // 16):
                        sl = pl.ds(j * 16, 16)
                        rb[k, sl] = rb[k, sl] * a
                return ()

            lax.fori_loop(0, K // 16, group, ())

        def step(c, b, first):
            bn = (b + 1) % 3
            wait_gather(b)
            wait_idx(bn)
            if not first:
                wait_scatter(bn)

            @pl.when(c + 1 < NCHUNKS)
            def _g():
                issue_gather(bn)

            issue_idst(c + 1, bn)
            compute(b)
            issue_idx(c + 3, b)
            wait_idst(b)
            issue_scatter(b)

        issue_idx(0, 0)
        issue_idx(1, 1)
        issue_idx(2, 2)
        issue_idst(0, 0)
        wait_idx(0)
        issue_gather(0)

        step(0, 0, True)
        step(1, 1, True)
        step(2, 2, False)

        def triple(p, _):
            c0 = 3 * p
            step(c0, 0, False)
            step(c0 + 1, 1, False)
            step(c0 + 2, 2, False)
            return ()

        lax.fori_loop(1, NTRIPLES, triple, ())

        step(NCHUNKS - 2, 0, False)
        step(NCHUNKS - 1, 1, False)
        wait_scatter(0)
        wait_scatter(1)
        wait_idx(0)
        wait_idx(1)
        wait_idst(2)
        plsc.subcore_barrier()

        pltpu.sync_copy(acc.at[pl.ds(sid * RPT, RPT)],
                        out_hbm.at[cid, pl.ds(sid * RPT, RPT)])

        @pl.when(sid == NS - 1)
        def _copy_tail():
            pltpu.sync_copy(acc.at[pl.ds(NS * RPT, N - NS * RPT)],
                            out_hbm.at[cid, pl.ds(NS * RPT, N - NS * RPT)])

    return agg(x, src, dst, adj)


def _tc_body(p_ref, w_ref, o_ref):
    s = p_ref[0] + p_ref[1]
    h = lax.dot_general(s, w_ref[...], (((1,), (1,)), ((), ())),
                        preferred_element_type=jnp.float32,
                        precision=lax.Precision.HIGHEST)
    o_ref[...] = jnp.maximum(h, 0.0)


def _tc_combine_matmul_relu(partials, W):
    bm = 2000
    return pl.pallas_call(
        _tc_body,
        grid=(N // bm,),
        in_specs=[
            pl.BlockSpec((NC, bm, D), lambda i: (0, i, 0)),
            pl.BlockSpec((D, D), lambda i: (0, 0)),
        ],
        out_specs=pl.BlockSpec((bm, D), lambda i: (i, 0)),
        out_shape=jax.ShapeDtypeStruct((N, D), jnp.float32),
    )(partials, W)


def kernel(x, edge_index, adj_values, W):
    partials = _sc_aggregate(x, edge_index[1], edge_index[0], adj_values)
    return _tc_combine_matmul_relu(partials, W)


# speedup vs baseline: 1.8156x; 1.0172x over previous
import functools

import jax
import jax.numpy as jnp
from jax import lax
from jax.experimental import pallas as pl
from jax.experimental.pallas import tpu as pltpu
from jax.experimental.pallas import tpu_sc as plsc

N = 10000
E = 320000
D = 128

NC = 2
NS = 16
NW = NC * NS
EPT = E // NW
K = 80
KQ = K // 4
NCHUNKS = EPT // K
NTRIPLES = NCHUNKS // 3
RPT = 624
ZR = 16


def _sc_aggregate(x, src, dst, adj):
    mesh = plsc.VectorSubcoreMesh(core_axis_name="c", subcore_axis_name="s")

    @functools.partial(
        pl.kernel,
        out_type=jax.ShapeDtypeStruct((NC, N, D), jnp.float32),
        mesh=mesh,
        scratch_types=[
            pltpu.VMEM_SHARED((N, D), jnp.float32),
            pltpu.VMEM((K,), jnp.int32),
            pltpu.VMEM((K,), jnp.int32),
            pltpu.VMEM((K,), jnp.int32),
            pltpu.VMEM((K,), jnp.int32),
            pltpu.VMEM((K,), jnp.int32),
            pltpu.VMEM((K,), jnp.int32),
            pltpu.VMEM((K,), jnp.float32),
            pltpu.VMEM((K,), jnp.float32),
            pltpu.VMEM((K,), jnp.float32),
            pltpu.VMEM((K, D), jnp.float32),
            pltpu.VMEM((K, D), jnp.float32),
            pltpu.VMEM((K, D), jnp.float32),
            pltpu.VMEM((ZR, D), jnp.float32),
            pltpu.SemaphoreType.DMA,
            pltpu.SemaphoreType.DMA,
            pltpu.SemaphoreType.DMA,
            pltpu.SemaphoreType.DMA,
            pltpu.SemaphoreType.DMA,
            pltpu.SemaphoreType.DMA,
            pltpu.SemaphoreType.DMA,
            pltpu.SemaphoreType.DMA,
            pltpu.SemaphoreType.DMA,
            pltpu.SemaphoreType.DMA,
            pltpu.SemaphoreType.DMA,
            pltpu.SemaphoreType.DMA,
            pltpu.SemaphoreType.DMA,
            pltpu.SemaphoreType.DMA,
            pltpu.SemaphoreType.DMA,
        ],
    )
    def agg(x_hbm, src_hbm, dst_hbm, adj_hbm, out_hbm,
            acc, isrc0, isrc1, isrc2, idst0, idst1, idst2, av0, av1, av2,
            rows0, rows1, rows2, zbuf,
            isem0, isem1, isem2, dsem0, dsem1, dsem2,
            gsa0, gsa1, gsa2, gsb0, gsb1, gsb2, ssem0, ssem1, ssem2):
        cid = lax.axis_index("c")
        sid = lax.axis_index("s")
        wid = cid * NS + sid
        base0 = wid * EPT

        isrc = (isrc0, isrc1, isrc2)
        idst = (idst0, idst1, idst2)
        av = (av0, av1, av2)
        rows = (rows0, rows1, rows2)
        isem = (isem0, isem1, isem2)
        dsem = (dsem0, dsem1, dsem2)
        gsa = (gsa0, gsa1, gsa2)
        gsb = (gsb0, gsb1, gsb2)
        ssem = (ssem0, ssem1, ssem2)

        def GSPLIT(b):
            return ((0, 24, gsa[b]), (24, 24, gsb[b]),
                    (48, 16, gsa[b]), (64, 16, gsb[b]))

        def _base(c):
            return jnp.minimum(base0 + c * K, E - K)

        def issue_idx(c, b):
            base = _base(c)
            pltpu.async_copy(src_hbm.at[pl.ds(base, K)], isrc[b], isem[b])
            pltpu.async_copy(adj_hbm.at[pl.ds(base, K)], av[b], isem[b])

        def wait_idx(b):
            pltpu.make_async_copy(src_hbm.at[pl.ds(0, K)], isrc[b], isem[b]).wait()
            pltpu.make_async_copy(adj_hbm.at[pl.ds(0, K)], av[b], isem[b]).wait()

        def issue_idst(c, b):
            pltpu.async_copy(dst_hbm.at[pl.ds(_base(c), K)], idst[b], dsem[b])

        def wait_idst(b):
            pltpu.make_async_copy(dst_hbm.at[pl.ds(0, K)], idst[b], dsem[b]).wait()

        def issue_gather(b):
            for o, n, sem in GSPLIT(b):
                pltpu.async_copy(x_hbm.at[isrc[b].at[pl.ds(o, n)]],
                                 rows[b].at[pl.ds(o, n)], sem)

        def wait_gather(b):
            for o, n, sem in GSPLIT(b):
                pltpu.make_async_copy(x_hbm.at[isrc[b].at[pl.ds(o, n)]],
                                      rows[b].at[pl.ds(o, n)], sem).wait()

        def issue_scatter(b):
            pltpu.async_copy(rows[b], acc.at[idst[b]], ssem[b], add=True)

        def wait_scatter(b):
            pltpu.make_async_copy(rows[b], acc.at[idst[b]], ssem[b]).wait()

        def compute(b):
            rb, ab = rows[b], av[b]

            def group(t, _):
                a16 = ab[pl.ds(t * 16, 16)]
                for i in range(16):
                    a = jnp.broadcast_to(a16[i], (16,))
                    k = t * 16 + i
                    for j in range(D // 16):
                        sl = pl.ds(j * 16, 16)
                        rb[k, sl] = rb[k, sl] * a
                return ()

            lax.fori_loop(0, K // 16, group, ())

        def step(c, b, first):
            bn = (b + 1) % 3
            wait_gather(b)
            wait_idx(bn)
            if not first:
                wait_scatter(bn)

            @pl.when(c + 1 < NCHUNKS)
            def _g():
                issue_gather(bn)

            issue_idst(c + 1, bn)
            compute(b)
            issue_idx(c + 3, b)
            wait_idst(b)
            issue_scatter(b)

        issue_idx(0, 0)
        issue_idx(1, 1)
        issue_idx(2, 2)
        issue_idst(0, 0)
        wait_idx(0)
        issue_gather(0)

        zv = jnp.zeros((16,), jnp.float32)

        def zrow(i, _):
            for j in range(D // 16):
                zbuf[i, pl.ds(j * 16, 16)] = zv
            return ()

        lax.fori_loop(0, ZR, zrow, ())

        def zcopy(i, _):
            pltpu.async_copy(zbuf, acc.at[pl.ds(sid * RPT + i * ZR, ZR)],
                             ssem2)
            return ()

        lax.fori_loop(0, RPT // ZR, zcopy, ())

        @pl.when(sid == NS - 1)
        def _zero_tail():
            pltpu.async_copy(zbuf, acc.at[pl.ds(NS * RPT, ZR)], ssem2)

        def zdrain(i, _):
            pltpu.make_async_copy(zbuf, acc.at[pl.ds(0, ZR)], ssem2).wait()
            return ()

        lax.fori_loop(0, RPT // ZR, zdrain, ())

        @pl.when(sid == NS - 1)
        def _zero_tail_drain():
            pltpu.make_async_copy(zbuf, acc.at[pl.ds(0, ZR)], ssem2).wait()

        plsc.subcore_barrier()

        step(0, 0, True)
        step(1, 1, True)
        step(2, 2, False)

        def triple(p, _):
            c0 = 3 * p
            step(c0, 0, False)
            step(c0 + 1, 1, False)
            step(c0 + 2, 2, False)
            return ()

        lax.fori_loop(1, NTRIPLES, triple, ())

        step(NCHUNKS - 2, 0, False)
        step(NCHUNKS - 1, 1, False)
        wait_scatter(0)
        wait_scatter(1)
        wait_idx(0)
        wait_idx(1)
        wait_idst(2)
        plsc.subcore_barrier()

        pltpu.sync_copy(acc.at[pl.ds(sid * RPT, RPT)],
                        out_hbm.at[cid, pl.ds(sid * RPT, RPT)])

        @pl.when(sid == NS - 1)
        def _copy_tail():
            pltpu.sync_copy(acc.at[pl.ds(NS * RPT, N - NS * RPT)],
                            out_hbm.at[cid, pl.ds(NS * RPT, N - NS * RPT)])

    return agg(x, src, dst, adj)


def _tc_body(p_ref, w_ref, o_ref):
    s = p_ref[0] + p_ref[1]
    h = lax.dot_general(s, w_ref[...], (((1,), (1,)), ((), ())),
                        preferred_element_type=jnp.float32)
    o_ref[...] = jnp.maximum(h, 0.0)


def _tc_combine_matmul_relu(partials, W):
    bm = 2000
    return pl.pallas_call(
        _tc_body,
        grid=(N // bm,),
        in_specs=[
            pl.BlockSpec((NC, bm, D), lambda i: (0, i, 0)),
            pl.BlockSpec((D, D), lambda i: (0, 0)),
        ],
        out_specs=pl.BlockSpec((bm, D), lambda i: (i, 0)),
        out_shape=jax.ShapeDtypeStruct((N, D), jnp.float32),
    )(partials, W)


def kernel(x, edge_index, adj_values, W):
    partials = _sc_aggregate(x, edge_index[1], edge_index[0], adj_values)
    return _tc_combine_matmul_relu(partials, W)


# speedup vs baseline: 1.9333x; 1.0648x over previous
import functools

import jax
import jax.numpy as jnp
from jax import lax
from jax.experimental import pallas as pl
from jax.experimental.pallas import tpu as pltpu
from jax.experimental.pallas import tpu_sc as plsc

N = 10000
E = 320000
D = 128

NC = 2
NS = 16
NW = NC * NS
EPT = E // NW
K = 80
KQ = K // 4
NCHUNKS = EPT // K
NTRIPLES = NCHUNKS // 3
RPT = 624
ZR = 16


def _sc_aggregate(x, ei_flat, adj):
    mesh = plsc.VectorSubcoreMesh(core_axis_name="c", subcore_axis_name="s")

    @functools.partial(
        pl.kernel,
        out_type=jax.ShapeDtypeStruct((NC, N, D), jnp.float32),
        mesh=mesh,
        scratch_types=[
            pltpu.VMEM_SHARED((N, D), jnp.float32),
            pltpu.VMEM((K,), jnp.int32),
            pltpu.VMEM((K,), jnp.int32),
            pltpu.VMEM((K,), jnp.int32),
            pltpu.VMEM((K,), jnp.int32),
            pltpu.VMEM((K,), jnp.int32),
            pltpu.VMEM((K,), jnp.int32),
            pltpu.VMEM((K,), jnp.float32),
            pltpu.VMEM((K,), jnp.float32),
            pltpu.VMEM((K,), jnp.float32),
            pltpu.VMEM((K, D), jnp.float32),
            pltpu.VMEM((K, D), jnp.float32),
            pltpu.VMEM((K, D), jnp.float32),
            pltpu.VMEM((ZR, D), jnp.float32),
            pltpu.SemaphoreType.DMA,
            pltpu.SemaphoreType.DMA,
            pltpu.SemaphoreType.DMA,
            pltpu.SemaphoreType.DMA,
            pltpu.SemaphoreType.DMA,
            pltpu.SemaphoreType.DMA,
            pltpu.SemaphoreType.DMA,
            pltpu.SemaphoreType.DMA,
            pltpu.SemaphoreType.DMA,
            pltpu.SemaphoreType.DMA,
            pltpu.SemaphoreType.DMA,
            pltpu.SemaphoreType.DMA,
            pltpu.SemaphoreType.DMA,
            pltpu.SemaphoreType.DMA,
            pltpu.SemaphoreType.DMA,
        ],
    )
    def agg(x_hbm, ei_hbm, adj_hbm, out_hbm,
            acc, isrc0, isrc1, isrc2, idst0, idst1, idst2, av0, av1, av2,
            rows0, rows1, rows2, zbuf,
            isem0, isem1, isem2, dsem0, dsem1, dsem2,
            gsa0, gsa1, gsa2, gsb0, gsb1, gsb2, ssem0, ssem1, ssem2):
        cid = lax.axis_index("c")
        sid = lax.axis_index("s")
        wid = cid * NS + sid
        base0 = wid * EPT

        isrc = (isrc0, isrc1, isrc2)
        idst = (idst0, idst1, idst2)
        av = (av0, av1, av2)
        rows = (rows0, rows1, rows2)
        isem = (isem0, isem1, isem2)
        dsem = (dsem0, dsem1, dsem2)
        gsa = (gsa0, gsa1, gsa2)
        gsb = (gsb0, gsb1, gsb2)
        ssem = (ssem0, ssem1, ssem2)

        def GSPLIT(b):
            return ((0, 24, gsa[b]), (24, 24, gsb[b]),
                    (48, 16, gsa[b]), (64, 16, gsb[b]))

        def _base(c):
            return jnp.minimum(base0 + c * K, E - K)

        def issue_idx(c, b):
            base = _base(c)
            pltpu.async_copy(ei_hbm.at[pl.ds(E + base, K)], isrc[b], isem[b])
            pltpu.async_copy(adj_hbm.at[pl.ds(base, K)], av[b], isem[b])

        def wait_idx(b):
            pltpu.make_async_copy(ei_hbm.at[pl.ds(0, K)], isrc[b], isem[b]).wait()
            pltpu.make_async_copy(adj_hbm.at[pl.ds(0, K)], av[b], isem[b]).wait()

        def issue_idst(c, b):
            pltpu.async_copy(ei_hbm.at[pl.ds(_base(c), K)], idst[b], dsem[b])

        def wait_idst(b):
            pltpu.make_async_copy(ei_hbm.at[pl.ds(0, K)], idst[b], dsem[b]).wait()

        def issue_gather(b):
            for o, n, sem in GSPLIT(b):
                pltpu.async_copy(x_hbm.at[isrc[b].at[pl.ds(o, n)]],
                                 rows[b].at[pl.ds(o, n)], sem)

        def wait_gather(b):
            for o, n, sem in GSPLIT(b):
                pltpu.make_async_copy(x_hbm.at[isrc[b].at[pl.ds(o, n)]],
                                      rows[b].at[pl.ds(o, n)], sem).wait()

        def issue_scatter(b):
            pltpu.async_copy(rows[b], acc.at[idst[b]], ssem[b], add=True)

        def wait_scatter(b):
            pltpu.make_async_copy(rows[b], acc.at[idst[b]], ssem[b]).wait()

        def compute(b):
            rb, ab = rows[b], av[b]

            def group(t, _):
                a16 = ab[pl.ds(t * 16, 16)]
                for i in range(16):
                    a = jnp.broadcast_to(a16[i], (16,))
                    k = t * 16 + i
                    for j in range(D // 16):
                        sl = pl.ds(j * 16, 16)
                        rb[k, sl] = rb[k, sl] * a
                return ()

            lax.fori_loop(0, K // 16, group, ())

        def step(c, b, first):
            bn = (b + 1) % 3
            wait_gather(b)
            wait_idx(bn)
            if not first:
                wait_scatter(bn)

            @pl.when(c + 1 < NCHUNKS)
            def _g():
                issue_gather(bn)

            issue_idst(c + 1, bn)
            compute(b)
            issue_idx(c + 3, b)
            wait_idst(b)
            issue_scatter(b)

        issue_idx(0, 0)
        issue_idx(1, 1)
        issue_idx(2, 2)
        issue_idst(0, 0)
        wait_idx(0)
        issue_gather(0)

        zv = jnp.zeros((16,), jnp.float32)

        def zrow(i, _):
            for j in range(D // 16):
                zbuf[i, pl.ds(j * 16, 16)] = zv
            return ()

        lax.fori_loop(0, ZR, zrow, ())

        def zcopy(i, _):
            pltpu.async_copy(zbuf, acc.at[pl.ds(sid * RPT + i * ZR, ZR)],
                             ssem2)
            return ()

        lax.fori_loop(0, RPT // ZR, zcopy, ())

        @pl.when(sid == NS - 1)
        def _zero_tail():
            pltpu.async_copy(zbuf, acc.at[pl.ds(NS * RPT, ZR)], ssem2)

        def zdrain(i, _):
            pltpu.make_async_copy(zbuf, acc.at[pl.ds(0, ZR)], ssem2).wait()
            return ()

        lax.fori_loop(0, RPT // ZR, zdrain, ())

        @pl.when(sid == NS - 1)
        def _zero_tail_drain():
            pltpu.make_async_copy(zbuf, acc.at[pl.ds(0, ZR)], ssem2).wait()

        plsc.subcore_barrier()

        step(0, 0, True)
        step(1, 1, True)
        step(2, 2, False)

        def triple(p, _):
            c0 = 3 * p
            step(c0, 0, False)
            step(c0 + 1, 1, False)
            step(c0 + 2, 2, False)
            return ()

        lax.fori_loop(1, NTRIPLES, triple, ())

        step(NCHUNKS - 2, 0, False)
        step(NCHUNKS - 1, 1, False)
        wait_scatter(0)
        wait_scatter(1)
        wait_idx(0)
        wait_idx(1)
        wait_idst(2)
        plsc.subcore_barrier()

        pltpu.sync_copy(acc.at[pl.ds(sid * RPT, RPT)],
                        out_hbm.at[cid, pl.ds(sid * RPT, RPT)])

        @pl.when(sid == NS - 1)
        def _copy_tail():
            pltpu.sync_copy(acc.at[pl.ds(NS * RPT, N - NS * RPT)],
                            out_hbm.at[cid, pl.ds(NS * RPT, N - NS * RPT)])

    return agg(x, ei_flat, adj)


def _tc_body(p_ref, w_ref, o_ref):
    s = p_ref[0] + p_ref[1]
    h = lax.dot_general(s, w_ref[...], (((1,), (1,)), ((), ())),
                        preferred_element_type=jnp.float32)
    o_ref[...] = jnp.maximum(h, 0.0)


def _tc_combine_matmul_relu(partials, W):
    bm = 2000
    return pl.pallas_call(
        _tc_body,
        grid=(N // bm,),
        in_specs=[
            pl.BlockSpec((NC, bm, D), lambda i: (0, i, 0)),
            pl.BlockSpec((D, D), lambda i: (0, 0)),
        ],
        out_specs=pl.BlockSpec((bm, D), lambda i: (i, 0)),
        out_shape=jax.ShapeDtypeStruct((N, D), jnp.float32),
    )(partials, W)


def kernel(x, edge_index, adj_values, W):
    partials = _sc_aggregate(x, edge_index.reshape(2 * E), adj_values)
    return _tc_combine_matmul_relu(partials, W)
